# Initial kernel scaffold; baseline (speedup 1.0000x reference)
import jax
import jax.numpy as jnp
from jax.experimental import pallas as pl


def kernel(src_features, src_labels, tgt_features, device):
    raise NotImplementedError("write your pallas kernel here")


# speedup vs baseline: 4.9451x; 4.9451x over previous
import jax
import jax.numpy as jnp
from jax.experimental import pallas as pl

N = 32
M = 4
MU = 16
K = 5
N1 = 8
N2 = 4
UNK = 10
HI = jax.lax.Precision.HIGHEST
F = jnp.float32


def _colb(v_row, eye, ones):
    diag = eye * v_row
    return jax.lax.dot_general(diag, ones, (((1,), (0,)), ((), ())),
                               precision=HI, preferred_element_type=F)


def _body(src_ref, tgtT_ref, lab_ref, loss_ref, ssi_ref, asg_ref, sum_ref):
    src = src_ref[...]
    tgtT = tgtT_ref[...]
    labels_row = lab_ref[...]

    an = jnp.sum(src * src, axis=1, keepdims=True)
    bn = jnp.sum(tgtT * tgtT, axis=0, keepdims=True)
    ab = jax.lax.dot_general(src, tgtT, (((1,), (0,)), ((), ())),
                             precision=HI, preferred_element_type=F)
    d2 = an + bn - 2.0 * ab
    sim = 1.0 / (jnp.sqrt(jnp.clip(d2, 0.0, None)) + 1.0)

    eye = (jax.lax.broadcasted_iota(jnp.int32, (N, N), 0)
           == jax.lax.broadcasted_iota(jnp.int32, (N, N), 1)).astype(F)
    ones = jnp.ones((N, N), F)
    ri = jax.lax.broadcasted_iota(jnp.int32, (N, N), 0)
    ci = jax.lax.broadcasted_iota(jnp.int32, (N, N), 1)

    labels_f = labels_row.astype(F)
    labels_colb = _colb(labels_f, eye, ones)

    r_all = jnp.zeros((N, N), jnp.int32)
    for k in range(N):
        rowk = sim[k:k + 1, :]
        ck = (rowk > sim) | ((rowk == sim) & (k < ri))
        r_all = r_all + ck.astype(jnp.int32)
    top = r_all < K

    best_key = jnp.zeros((1, N), jnp.int32)
    assigned = jnp.zeros((1, N), jnp.int32)
    for c in range(10):
        cnt = jnp.sum(jnp.where(top & (labels_colb == c), 1, 0),
                      axis=0, keepdims=True)
        key = cnt * 16 - c
        win = key > best_key
        best_key = jnp.where(win, key, best_key)
        assigned = jnp.where(win, c, assigned)

    line_max = jnp.max(sim, axis=0, keepdims=True)
    colsum = jnp.sum(sim, axis=0, keepdims=True)
    lm_colb = _colb(line_max, eye, ones)
    rank1 = jnp.sum(((lm_colb < line_max)
                     | ((lm_colb == line_max) & (ri < ci))).astype(jnp.int32),
                    axis=0, keepdims=True)
    in_cut = rank1 < N1
    cs_colb = _colb(colsum, eye, ones)
    r1_colb = _colb(rank1.astype(F), eye, ones)
    in_cut_colb = _colb(in_cut.astype(F), eye, ones) > 0.5
    rank1_f = rank1.astype(F)
    rank2 = jnp.sum((in_cut_colb
                     & ((cs_colb < colsum)
                        | ((cs_colb == colsum) & (r1_colb < rank1_f)))
                     ).astype(jnp.int32), axis=0, keepdims=True)
    selected = in_cut & (rank2 < N2)
    assigned = jnp.where(selected, UNK, assigned)

    selb = jnp.broadcast_to(selected, (4, N))
    r2b = jnp.broadcast_to(rank2, (4, N))
    mrow = jax.lax.broadcasted_iota(jnp.int32, (4, N), 0)
    jcol = jax.lax.broadcasted_iota(jnp.int32, (4, N), 1)
    ssi = jnp.sum(jnp.where(selb & (r2b == mrow), jcol, 0),
                  axis=1, keepdims=True)

    mask_same = labels_colb == assigned.astype(F)
    r_same = jnp.zeros((N, N), jnp.int32)
    for k in range(N):
        rowk = sim[k:k + 1, :]
        ck = (rowk > sim) | ((rowk == sim) & (k < ri))
        mk = mask_same[k:k + 1, :]
        r_same = r_same + (ck & mk).astype(jnp.int32)
    r_diff = r_all - r_same

    cnt_same = jnp.sum(mask_same.astype(jnp.int32), axis=0, keepdims=True)
    cnt_diff = N - cnt_same
    take_s = jnp.clip(jnp.minimum(cnt_same, M), 1, None).astype(F)
    take_d = jnp.clip(jnp.minimum(cnt_diff, M), 1, None).astype(F)
    inc_s = mask_same & (r_same.astype(F) < take_s)
    inc_d = (~mask_same) & (r_diff.astype(F) < take_d)
    tm_s = jnp.sum(jnp.where(inc_s, sim, 0.0), axis=0, keepdims=True) / take_s
    tm_d = jnp.sum(jnp.where(inc_d, sim, 0.0), axis=0, keepdims=True) / take_d
    scores = tm_s / tm_d

    kept = ~selected
    s_colb = _colb(scores, eye, ones)
    kept_colb = _colb(kept.astype(F), eye, ones) > 0.5
    rank_sc = jnp.sum((kept_colb
                       & ((s_colb > scores)
                          | ((s_colb == scores) & (ri < ci)))
                       ).astype(jnp.int32), axis=0, keepdims=True)
    filt = kept & (rank_sc < MU)

    e = jnp.exp(sim)
    filt_f = filt.astype(F)
    denom_flt = jnp.sum(e * filt_f, axis=1, keepdims=True)
    numer = jnp.sum(e * filt_f * mask_same.astype(F), axis=1, keepdims=True)
    sel_f = selected.astype(F)
    unk = jax.lax.dot_general(sel_f, e, (((1,), (0,)), ((), ())),
                              precision=HI, preferred_element_type=F)
    unk_colb = _colb(unk, eye, ones)[:, 0:1]
    denom = denom_flt + unk_colb
    valid = numer > 0.0
    contr = numer / denom
    logc = jnp.log(jnp.where(valid, contr, 1.0))
    nvalid = jnp.sum(valid.astype(F))
    loss = -jnp.sum(logc) / nvalid

    loss_ref[...] = loss.reshape(1, 1)
    ssi_ref[...] = ssi
    asg_ref[...] = assigned
    sum_ref[...] = colsum


def kernel(src_features, src_labels, tgt_features, device=0):
    tgtT = tgt_features.T
    lab = src_labels.reshape(1, N)
    loss, ssi, asg, sl = pl.pallas_call(
        _body,
        out_shape=[
            jax.ShapeDtypeStruct((1, 1), F),
            jax.ShapeDtypeStruct((4, 1), jnp.int32),
            jax.ShapeDtypeStruct((1, N), jnp.int32),
            jax.ShapeDtypeStruct((1, N), F),
        ],
    )(src_features, tgtT, lab)
    return loss.reshape(()), ssi.reshape(N2), asg.reshape(N), sl.reshape(N)


# speedup vs baseline: 5.1214x; 1.0356x over previous
import jax
import jax.numpy as jnp
from jax.experimental import pallas as pl

N = 32
M = 4
MU = 16
K = 5
N1 = 8
N2 = 4
UNK = 10
HI = jax.lax.Precision.HIGHEST
F = jnp.float32


def _colb(v_row, eye, ones):
    diag = eye * v_row
    return jax.lax.dot_general(diag, ones, (((1,), (0,)), ((), ())),
                               precision=HI, preferred_element_type=F)


def _body(src_ref, tgtT_ref, lab_ref, loss_ref, ssi_ref, asg_ref, sum_ref):
    src = src_ref[...]
    tgtT = tgtT_ref[...]
    labels_row = lab_ref[...]

    an = jnp.sum(src * src, axis=1, keepdims=True)
    bn = jnp.sum(tgtT * tgtT, axis=0, keepdims=True)
    ab = jax.lax.dot_general(src, tgtT, (((1,), (0,)), ((), ())),
                             precision=jax.lax.Precision.DEFAULT,
                             preferred_element_type=F)
    d2 = an + bn - 2.0 * ab
    sim = 1.0 / (jnp.sqrt(jnp.clip(d2, 0.0, None)) + 1.0)

    eye = (jax.lax.broadcasted_iota(jnp.int32, (N, N), 0)
           == jax.lax.broadcasted_iota(jnp.int32, (N, N), 1)).astype(F)
    ones = jnp.ones((N, N), F)
    ri = jax.lax.broadcasted_iota(jnp.int32, (N, N), 0)
    ci = jax.lax.broadcasted_iota(jnp.int32, (N, N), 1)

    labels_f = labels_row.astype(F)
    labels_colb = _colb(labels_f, eye, ones)

    r_all = jnp.zeros((N, N), jnp.int32)
    for k in range(N):
        rowk = sim[k:k + 1, :]
        ck = (rowk > sim) | ((rowk == sim) & (k < ri))
        r_all = r_all + ck.astype(jnp.int32)
    top = r_all < K

    best_key = jnp.zeros((1, N), jnp.int32)
    assigned = jnp.zeros((1, N), jnp.int32)
    for c in range(10):
        cnt = jnp.sum(jnp.where(top & (labels_colb == c), 1, 0),
                      axis=0, keepdims=True)
        key = cnt * 16 - c
        win = key > best_key
        best_key = jnp.where(win, key, best_key)
        assigned = jnp.where(win, c, assigned)

    line_max = jnp.max(sim, axis=0, keepdims=True)
    colsum = jnp.sum(sim, axis=0, keepdims=True)
    lm_colb = _colb(line_max, eye, ones)
    rank1 = jnp.sum(((lm_colb < line_max)
                     | ((lm_colb == line_max) & (ri < ci))).astype(jnp.int32),
                    axis=0, keepdims=True)
    in_cut = rank1 < N1
    cs_colb = _colb(colsum, eye, ones)
    r1_colb = _colb(rank1.astype(F), eye, ones)
    in_cut_colb = _colb(in_cut.astype(F), eye, ones) > 0.5
    rank1_f = rank1.astype(F)
    rank2 = jnp.sum((in_cut_colb
                     & ((cs_colb < colsum)
                        | ((cs_colb == colsum) & (r1_colb < rank1_f)))
                     ).astype(jnp.int32), axis=0, keepdims=True)
    selected = in_cut & (rank2 < N2)
    assigned = jnp.where(selected, UNK, assigned)

    selb = jnp.broadcast_to(selected, (4, N))
    r2b = jnp.broadcast_to(rank2, (4, N))
    mrow = jax.lax.broadcasted_iota(jnp.int32, (4, N), 0)
    jcol = jax.lax.broadcasted_iota(jnp.int32, (4, N), 1)
    ssi = jnp.sum(jnp.where(selb & (r2b == mrow), jcol, 0),
                  axis=1, keepdims=True)

    mask_same = labels_colb == assigned.astype(F)
    r_same = jnp.zeros((N, N), jnp.int32)
    for k in range(N):
        rowk = sim[k:k + 1, :]
        ck = (rowk > sim) | ((rowk == sim) & (k < ri))
        mk = mask_same[k:k + 1, :]
        r_same = r_same + (ck & mk).astype(jnp.int32)
    r_diff = r_all - r_same

    cnt_same = jnp.sum(mask_same.astype(jnp.int32), axis=0, keepdims=True)
    cnt_diff = N - cnt_same
    take_s = jnp.clip(jnp.minimum(cnt_same, M), 1, None).astype(F)
    take_d = jnp.clip(jnp.minimum(cnt_diff, M), 1, None).astype(F)
    inc_s = mask_same & (r_same.astype(F) < take_s)
    inc_d = (~mask_same) & (r_diff.astype(F) < take_d)
    tm_s = jnp.sum(jnp.where(inc_s, sim, 0.0), axis=0, keepdims=True) / take_s
    tm_d = jnp.sum(jnp.where(inc_d, sim, 0.0), axis=0, keepdims=True) / take_d
    scores = tm_s / tm_d

    kept = ~selected
    s_colb = _colb(scores, eye, ones)
    kept_colb = _colb(kept.astype(F), eye, ones) > 0.5
    rank_sc = jnp.sum((kept_colb
                       & ((s_colb > scores)
                          | ((s_colb == scores) & (ri < ci)))
                       ).astype(jnp.int32), axis=0, keepdims=True)
    filt = kept & (rank_sc < MU)

    e = jnp.exp(sim)
    filt_f = filt.astype(F)
    denom_flt = jnp.sum(e * filt_f, axis=1, keepdims=True)
    numer = jnp.sum(e * filt_f * mask_same.astype(F), axis=1, keepdims=True)
    sel_f = selected.astype(F)
    unk = jax.lax.dot_general(sel_f, e, (((1,), (0,)), ((), ())),
                              precision=HI, preferred_element_type=F)
    unk_colb = _colb(unk, eye, ones)[:, 0:1]
    denom = denom_flt + unk_colb
    valid = numer > 0.0
    contr = numer / denom
    logc = jnp.log(jnp.where(valid, contr, 1.0))
    nvalid = jnp.sum(valid.astype(F))
    loss = -jnp.sum(logc) / nvalid

    loss_ref[...] = loss.reshape(1, 1)
    ssi_ref[...] = ssi
    asg_ref[...] = assigned
    sum_ref[...] = colsum


def kernel(src_features, src_labels, tgt_features, device=0):
    tgtT = tgt_features.T
    lab = src_labels.reshape(1, N)
    loss, ssi, asg, sl = pl.pallas_call(
        _body,
        out_shape=[
            jax.ShapeDtypeStruct((1, 1), F),
            jax.ShapeDtypeStruct((4, 1), jnp.int32),
            jax.ShapeDtypeStruct((1, N), jnp.int32),
            jax.ShapeDtypeStruct((1, N), F),
        ],
    )(src_features, tgtT, lab)
    return loss.reshape(()), ssi.reshape(N2), asg.reshape(N), sl.reshape(N)


# speedup vs baseline: 7.3792x; 1.4409x over previous
import jax
import jax.numpy as jnp
from jax.experimental import pallas as pl

N = 32
M = 4
MU = 16
K = 5
N1 = 8
N2 = 4
UNK = 10
HI = jax.lax.Precision.HIGHEST
F = jnp.float32


def _colb(v_row, eye, ones):
    diag = eye * v_row
    return jax.lax.dot_general(diag, ones, (((1,), (0,)), ((), ())),
                               precision=HI, preferred_element_type=F)


def _body(src_ref, tgt_ref, lab_ref, loss_ref, ssi_ref, asg_ref, sum_ref):
    src = src_ref[...]
    tgt = tgt_ref[...]
    labels_row = lab_ref[...].reshape(1, N)

    an = jnp.sum(src * src, axis=1, keepdims=True)
    bn = jax.lax.dot_general(jnp.ones((1, src.shape[1]), F), tgt * tgt,
                             (((1,), (1,)), ((), ())),
                             precision=HI, preferred_element_type=F)
    ab = jax.lax.dot_general(src, tgt, (((1,), (1,)), ((), ())),
                             precision=jax.lax.Precision.DEFAULT,
                             preferred_element_type=F)
    d2 = an + bn - 2.0 * ab
    sim = 1.0 / (jnp.sqrt(jnp.clip(d2, 0.0, None)) + 1.0)

    eye = (jax.lax.broadcasted_iota(jnp.int32, (N, N), 0)
           == jax.lax.broadcasted_iota(jnp.int32, (N, N), 1)).astype(F)
    ones = jnp.ones((N, N), F)
    ri = jax.lax.broadcasted_iota(jnp.int32, (N, N), 0)
    ci = jax.lax.broadcasted_iota(jnp.int32, (N, N), 1)

    labels_f = labels_row.astype(F)
    labels_colb = _colb(labels_f, eye, ones)

    r_all = jnp.zeros((N, N), jnp.int32)
    for k in range(N):
        rowk = sim[k:k + 1, :]
        ck = (rowk > sim) | ((rowk == sim) & (k < ri))
        r_all = r_all + ck.astype(jnp.int32)
    top = r_all < K

    best_key = jnp.zeros((1, N), jnp.int32)
    assigned = jnp.zeros((1, N), jnp.int32)
    for c in range(10):
        cnt = jnp.sum(jnp.where(top & (labels_colb == c), 1, 0),
                      axis=0, keepdims=True)
        key = cnt * 16 - c
        win = key > best_key
        best_key = jnp.where(win, key, best_key)
        assigned = jnp.where(win, c, assigned)

    line_max = jnp.max(sim, axis=0, keepdims=True)
    colsum = jnp.sum(sim, axis=0, keepdims=True)
    lm_colb = _colb(line_max, eye, ones)
    rank1 = jnp.sum(((lm_colb < line_max)
                     | ((lm_colb == line_max) & (ri < ci))).astype(jnp.int32),
                    axis=0, keepdims=True)
    in_cut = rank1 < N1
    cs_colb = _colb(colsum, eye, ones)
    r1_colb = _colb(rank1.astype(F), eye, ones)
    in_cut_colb = _colb(in_cut.astype(F), eye, ones) > 0.5
    rank1_f = rank1.astype(F)
    rank2 = jnp.sum((in_cut_colb
                     & ((cs_colb < colsum)
                        | ((cs_colb == colsum) & (r1_colb < rank1_f)))
                     ).astype(jnp.int32), axis=0, keepdims=True)
    selected = in_cut & (rank2 < N2)
    assigned = jnp.where(selected, UNK, assigned)

    selb = jnp.broadcast_to(selected, (4, N))
    r2b = jnp.broadcast_to(rank2, (4, N))
    mrow = jax.lax.broadcasted_iota(jnp.int32, (4, N), 0)
    jcol = jax.lax.broadcasted_iota(jnp.int32, (4, N), 1)
    ssi = jnp.sum(jnp.where(selb & (r2b == mrow), jcol, 0),
                  axis=1, keepdims=True)

    mask_same = labels_colb == assigned.astype(F)
    r_same = jnp.zeros((N, N), jnp.int32)
    for k in range(N):
        rowk = sim[k:k + 1, :]
        ck = (rowk > sim) | ((rowk == sim) & (k < ri))
        mk = mask_same[k:k + 1, :]
        r_same = r_same + (ck & mk).astype(jnp.int32)
    r_diff = r_all - r_same

    cnt_same = jnp.sum(mask_same.astype(jnp.int32), axis=0, keepdims=True)
    cnt_diff = N - cnt_same
    take_s = jnp.clip(jnp.minimum(cnt_same, M), 1, None).astype(F)
    take_d = jnp.clip(jnp.minimum(cnt_diff, M), 1, None).astype(F)
    inc_s = mask_same & (r_same.astype(F) < take_s)
    inc_d = (~mask_same) & (r_diff.astype(F) < take_d)
    tm_s = jnp.sum(jnp.where(inc_s, sim, 0.0), axis=0, keepdims=True) / take_s
    tm_d = jnp.sum(jnp.where(inc_d, sim, 0.0), axis=0, keepdims=True) / take_d
    scores = tm_s / tm_d

    kept = ~selected
    s_colb = _colb(scores, eye, ones)
    kept_colb = _colb(kept.astype(F), eye, ones) > 0.5
    rank_sc = jnp.sum((kept_colb
                       & ((s_colb > scores)
                          | ((s_colb == scores) & (ri < ci)))
                       ).astype(jnp.int32), axis=0, keepdims=True)
    filt = kept & (rank_sc < MU)

    e = jnp.exp(sim)
    filt_f = filt.astype(F)
    denom_flt = jnp.sum(e * filt_f, axis=1, keepdims=True)
    numer = jnp.sum(e * filt_f * mask_same.astype(F), axis=1, keepdims=True)
    sel_f = selected.astype(F)
    unk = jax.lax.dot_general(sel_f, e, (((1,), (0,)), ((), ())),
                              precision=HI, preferred_element_type=F)
    unk_colb = _colb(unk, eye, ones)[:, 0:1]
    denom = denom_flt + unk_colb
    valid = numer > 0.0
    contr = numer / denom
    logc = jnp.log(jnp.where(valid, contr, 1.0))
    nvalid = jnp.sum(valid.astype(F))
    loss = -jnp.sum(logc) / nvalid

    loss_ref[...] = loss.reshape(1, 1)
    ssi_ref[...] = ssi
    asg_ref[...] = assigned
    sum_ref[...] = colsum


def kernel(src_features, src_labels, tgt_features, device=0):
    loss, ssi, asg, sl = pl.pallas_call(
        _body,
        out_shape=[
            jax.ShapeDtypeStruct((1, 1), F),
            jax.ShapeDtypeStruct((4, 1), jnp.int32),
            jax.ShapeDtypeStruct((1, N), jnp.int32),
            jax.ShapeDtypeStruct((1, N), F),
        ],
    )(src_features, tgt_features, src_labels)
    return loss.reshape(()), ssi.reshape(N2), asg.reshape(N), sl.reshape(N)


# speedup vs baseline: 9.8432x; 1.3339x over previous
import jax
import jax.numpy as jnp
from jax.experimental import pallas as pl

N = 32
M = 4
MU = 16
K = 5
N1 = 8
N2 = 4
UNK = 10
HI = jax.lax.Precision.HIGHEST
F = jnp.float32


def _colb(v_row, eye, ones):
    diag = eye * v_row
    return jax.lax.dot_general(diag, ones, (((1,), (0,)), ((), ())),
                               precision=HI, preferred_element_type=F)


def _body(src_ref, tgt_ref, lab_ref, loss_ref, ssi_ref, asg_ref, sum_ref):
    src = src_ref[...]
    tgt = tgt_ref[...]
    labels_row = lab_ref[...].reshape(1, N)

    an = jnp.sum(src * src, axis=1, keepdims=True)
    bn = jax.lax.dot_general(jnp.ones((1, src.shape[1]), F), tgt * tgt,
                             (((1,), (1,)), ((), ())),
                             precision=HI, preferred_element_type=F)
    ab = jax.lax.dot_general(src, tgt, (((1,), (1,)), ((), ())),
                             precision=jax.lax.Precision.DEFAULT,
                             preferred_element_type=F)
    d2 = an + bn - 2.0 * ab
    sim = 1.0 / (jnp.sqrt(jnp.clip(d2, 0.0, None)) + 1.0)

    eye = (jax.lax.broadcasted_iota(jnp.int32, (N, N), 0)
           == jax.lax.broadcasted_iota(jnp.int32, (N, N), 1)).astype(F)
    ones = jnp.ones((N, N), F)
    ri = jax.lax.broadcasted_iota(jnp.int32, (N, N), 0)
    ci = jax.lax.broadcasted_iota(jnp.int32, (N, N), 1)

    labels_f = labels_row.astype(F)
    labels_colb = _colb(labels_f, eye, ones)

    r_all = jnp.zeros((N, N), jnp.int32)
    for k in range(N):
        rowk = sim[k:k + 1, :]
        ck = (rowk > sim) | ((rowk == sim) & (k < ri))
        r_all = r_all + ck.astype(jnp.int32)
    top = r_all < K

    best_key = jnp.zeros((1, N), jnp.int32)
    assigned = jnp.zeros((1, N), jnp.int32)
    for c in range(10):
        cnt = jnp.sum(jnp.where(top & (labels_colb == c), 1, 0),
                      axis=0, keepdims=True)
        key = cnt * 16 - c
        win = key > best_key
        best_key = jnp.where(win, key, best_key)
        assigned = jnp.where(win, c, assigned)

    line_max = jnp.max(sim, axis=0, keepdims=True)
    colsum = jnp.sum(sim, axis=0, keepdims=True)
    lm_colb = _colb(line_max, eye, ones)
    rank1 = jnp.sum(((lm_colb < line_max)
                     | ((lm_colb == line_max) & (ri < ci))).astype(jnp.int32),
                    axis=0, keepdims=True)
    in_cut = rank1 < N1
    cs_colb = _colb(colsum, eye, ones)
    r1_colb = _colb(rank1.astype(F), eye, ones)
    in_cut_colb = _colb(in_cut.astype(F), eye, ones) > 0.5
    rank1_f = rank1.astype(F)
    rank2 = jnp.sum((in_cut_colb
                     & ((cs_colb < colsum)
                        | ((cs_colb == colsum) & (r1_colb < rank1_f)))
                     ).astype(jnp.int32), axis=0, keepdims=True)
    selected = in_cut & (rank2 < N2)
    assigned = jnp.where(selected, UNK, assigned)

    sel_colb = _colb(selected.astype(F), eye, ones)[:, 0:4]
    r2_colb = _colb(rank2.astype(F), eye, ones)[:, 0:4]
    m4 = jax.lax.broadcasted_iota(jnp.int32, (N, 4), 1).astype(F)
    ri4 = jax.lax.broadcasted_iota(jnp.int32, (N, 4), 0).astype(F)
    w4 = jnp.where((sel_colb > 0.5) & (r2_colb == m4), ri4, 0.0)
    ssi = jnp.sum(w4, axis=0, keepdims=True).astype(jnp.int32)

    mask_same = labels_colb == assigned.astype(F)
    r_same = jnp.zeros((N, N), jnp.int32)
    for k in range(N):
        rowk = sim[k:k + 1, :]
        ck = (rowk > sim) | ((rowk == sim) & (k < ri))
        mk = mask_same[k:k + 1, :]
        r_same = r_same + (ck & mk).astype(jnp.int32)
    r_diff = r_all - r_same

    cnt_same = jnp.sum(mask_same.astype(jnp.int32), axis=0, keepdims=True)
    cnt_diff = N - cnt_same
    take_s = jnp.clip(jnp.minimum(cnt_same, M), 1, None).astype(F)
    take_d = jnp.clip(jnp.minimum(cnt_diff, M), 1, None).astype(F)
    inc_s = mask_same & (r_same.astype(F) < take_s)
    inc_d = (~mask_same) & (r_diff.astype(F) < take_d)
    tm_s = jnp.sum(jnp.where(inc_s, sim, 0.0), axis=0, keepdims=True) / take_s
    tm_d = jnp.sum(jnp.where(inc_d, sim, 0.0), axis=0, keepdims=True) / take_d
    scores = tm_s / tm_d

    kept = ~selected
    s_colb = _colb(scores, eye, ones)
    kept_colb = _colb(kept.astype(F), eye, ones) > 0.5
    rank_sc = jnp.sum((kept_colb
                       & ((s_colb > scores)
                          | ((s_colb == scores) & (ri < ci)))
                       ).astype(jnp.int32), axis=0, keepdims=True)
    filt = kept & (rank_sc < MU)

    e = jnp.exp(sim)
    filt_f = filt.astype(F)
    denom_flt = jnp.sum(e * filt_f, axis=1, keepdims=True)
    numer = jnp.sum(e * filt_f * mask_same.astype(F), axis=1, keepdims=True)
    sel_f = selected.astype(F)
    unk = jax.lax.dot_general(sel_f, e, (((1,), (0,)), ((), ())),
                              precision=HI, preferred_element_type=F)
    unk_colb = _colb(unk, eye, ones)[:, 0:1]
    denom = denom_flt + unk_colb
    valid = numer > 0.0
    contr = numer / denom
    logc = jnp.log(jnp.where(valid, contr, 1.0))
    nvalid = jnp.sum(valid.astype(F))
    loss = -jnp.sum(logc) / nvalid

    loss_ref[...] = loss.reshape(1)
    ssi_ref[...] = ssi.reshape(4)
    asg_ref[...] = assigned.reshape(N)
    sum_ref[...] = colsum.reshape(N)


def kernel(src_features, src_labels, tgt_features, device=0):
    loss, ssi, asg, sl = pl.pallas_call(
        _body,
        out_shape=[
            jax.ShapeDtypeStruct((1,), F),
            jax.ShapeDtypeStruct((4,), jnp.int32),
            jax.ShapeDtypeStruct((N,), jnp.int32),
            jax.ShapeDtypeStruct((N,), F),
        ],
    )(src_features, tgt_features, src_labels)
    return loss.reshape(()), ssi, asg, sl
